# baseline (speedup 1.0000x reference)
import jax
import jax.numpy as jnp
from jax.experimental import pallas as pl
from jax.experimental.pallas import tpu as pltpu


_R = 256
_K = 16
_P = 10


def _make_body(C, NP, R, D):
    def body(x_ref, pe_ref, o_ref, xbuf, pecache, insem, pesem, outsem):
        def in_copy(t):
            return pltpu.make_async_copy(
                x_ref.at[pl.ds(t * R, R), :], xbuf.at[t % _K], insem.at[t % _K])

        def out_copy(t):
            return pltpu.make_async_copy(
                xbuf.at[t % _K], o_ref.at[pl.ds(t * R, R), :], outsem.at[t % _K])

        pe_copies = [
            pltpu.make_async_copy(
                pe_ref.at[pl.ds(p * R, R), :], pecache.at[p], pesem.at[p])
            for p in range(NP)
        ]
        in_copy(0).start()
        pe_copies[0].start()
        for j in range(1, min(_P, C)):
            in_copy(j).start()
        for c in pe_copies[1:]:
            c.start()

        out_waited = [False] * C
        pe_waited = [False] * NP
        for t in range(C):
            slot = t % _K
            in_copy(t).wait()
            p = t % NP
            if not pe_waited[p]:
                pe_copies[p].wait()
                pe_waited[p] = True
            xbuf[slot] = xbuf[slot] + pecache[p]
            out_copy(t).start()
            j = t + _P
            if j < C:
                if j >= _K:
                    out_copy(j - _K).wait()
                    out_waited[j - _K] = True
                in_copy(j).start()
        for t in range(C):
            if not out_waited[t]:
                out_copy(t).wait()

    return body


def kernel(x, pe_table):
    B, S, D = x.shape
    pe = pe_table[:S]
    xf = x.reshape(B * S, D)
    R = _R if (B * S) % _R == 0 and S % _R == 0 else S
    C = (B * S) // R
    NP = S // R
    out = pl.pallas_call(
        _make_body(C, NP, R, D),
        in_specs=[
            pl.BlockSpec(memory_space=pl.ANY),
            pl.BlockSpec(memory_space=pl.ANY),
        ],
        out_specs=pl.BlockSpec(memory_space=pl.ANY),
        out_shape=jax.ShapeDtypeStruct((B * S, D), x.dtype),
        scratch_shapes=[
            pltpu.VMEM((_K, R, D), x.dtype),
            pltpu.VMEM((NP, R, D), x.dtype),
            pltpu.SemaphoreType.DMA((_K,)),
            pltpu.SemaphoreType.DMA((NP,)),
            pltpu.SemaphoreType.DMA((_K,)),
        ],
    )(xf, pe)
    return out.reshape(B, S, D)


# speedup vs baseline: 1.0062x; 1.0062x over previous
import jax
import jax.numpy as jnp
from jax.experimental import pallas as pl
from jax.experimental.pallas import tpu as pltpu


_R = 1024
_K = 6
_P = 3


def _make_body(C, NP, R, D):
    def body(x_ref, pe_ref, o_ref, xbuf, pecache, insem, pesem, outsem):
        def in_copy(t):
            return pltpu.make_async_copy(
                x_ref.at[pl.ds(t * R, R), :], xbuf.at[t % _K], insem.at[t % _K])

        def out_copy(t):
            return pltpu.make_async_copy(
                xbuf.at[t % _K], o_ref.at[pl.ds(t * R, R), :], outsem.at[t % _K])

        pe_copies = [
            pltpu.make_async_copy(
                pe_ref.at[pl.ds(p * R, R), :], pecache.at[p], pesem.at[p])
            for p in range(NP)
        ]
        in_copy(0).start()
        pe_copies[0].start()
        for j in range(1, min(_P, C)):
            in_copy(j).start()
        for c in pe_copies[1:]:
            c.start()

        out_waited = [False] * C
        pe_waited = [False] * NP
        for t in range(C):
            slot = t % _K
            in_copy(t).wait()
            p = t % NP
            if not pe_waited[p]:
                pe_copies[p].wait()
                pe_waited[p] = True
            xbuf[slot] = xbuf[slot] + pecache[p]
            out_copy(t).start()
            j = t + _P
            if j < C:
                if j >= _K:
                    out_copy(j - _K).wait()
                    out_waited[j - _K] = True
                in_copy(j).start()
        for t in range(C):
            if not out_waited[t]:
                out_copy(t).wait()

    return body


def kernel(x, pe_table):
    B, S, D = x.shape
    pe = pe_table[:S]
    xf = x.reshape(B * S, D)
    R = _R if (B * S) % _R == 0 and S % _R == 0 else S
    C = (B * S) // R
    NP = S // R
    out = pl.pallas_call(
        _make_body(C, NP, R, D),
        in_specs=[
            pl.BlockSpec(memory_space=pl.ANY),
            pl.BlockSpec(memory_space=pl.ANY),
        ],
        out_specs=pl.BlockSpec(memory_space=pl.ANY),
        out_shape=jax.ShapeDtypeStruct((B * S, D), x.dtype),
        scratch_shapes=[
            pltpu.VMEM((_K, R, D), x.dtype),
            pltpu.VMEM((NP, R, D), x.dtype),
            pltpu.SemaphoreType.DMA((_K,)),
            pltpu.SemaphoreType.DMA((NP,)),
            pltpu.SemaphoreType.DMA((_K,)),
        ],
    )(xf, pe)
    return out.reshape(B, S, D)
